# baseline (speedup 1.0000x reference)
import jax
import jax.numpy as jnp
from jax import lax
from jax.experimental import pallas as pl
from jax.experimental.pallas import tpu as pltpu
from jax.experimental.pallas import tpu_sc as plsc

N_TOKENS = 32768
INPUT_DIM = 768
NUM_EXPERTS = 8
NC, NS, L = 2, 16, 16
NW = NC * NS
TOK_PER_W = N_TOKENS // NW
CHUNK = 64
G = CHUNK // L
N_CHUNKS = TOK_PER_W // CHUNK


def _sc_body(x_hbm, w_hbm, b_hbm, o_hbm, xb0, xb1, wv, bv, obuf, sem0, sem1):
    wid = lax.axis_index("s") * NC + lax.axis_index("c")
    base = wid * TOK_PER_W

    pltpu.sync_copy(w_hbm, wv)
    pltpu.sync_copy(b_hbm, bv)

    lane = lax.iota(jnp.int32, L)
    rows = [lane + (g * L) for g in range(G)]
    e_idx = [jnp.full((L,), e, jnp.int32) for e in range(NUM_EXPERTS)]
    bbc = [bv[e] for e in range(NUM_EXPERTS)]

    bufs = [xb0, xb1]
    sems = [sem0, sem1]
    copies = [None, None]

    def start_copy(c):
        return pltpu.async_copy(
            x_hbm.at[pl.ds(base + c * CHUNK, CHUNK)], bufs[c % 2], sems[c % 2])

    copies[0] = start_copy(0)

    for c in range(N_CHUNKS):
        copies[c % 2].wait()
        if c + 1 < N_CHUNKS:
            copies[(c + 1) % 2] = start_copy(c + 1)
        xref = bufs[c % 2]

        def dbody(d, accs):
            accs = list(accs)
            dcol = jnp.full((L,), d, jnp.int32)
            wbc = [plsc.load_gather(wv, [dcol, e_idx[e]])
                   for e in range(NUM_EXPERTS)]
            for g in range(G):
                xv = plsc.load_gather(xref, [rows[g], dcol])
                for e in range(NUM_EXPERTS):
                    k = g * NUM_EXPERTS + e
                    accs[k] = accs[k] + xv * wbc[e]
            return tuple(accs)

        init = tuple(bbc[e] for g in range(G) for e in range(NUM_EXPERTS))
        accs = lax.fori_loop(0, INPUT_DIM, dbody, init)

        for g in range(G):
            a = [accs[g * NUM_EXPERTS + e] for e in range(NUM_EXPERTS)]
            m = a[0]
            for e in range(1, NUM_EXPERTS):
                m = jnp.maximum(m, a[e])
            ex = [jnp.exp(v - m) for v in a]
            s = ex[0]
            for e in range(1, NUM_EXPERTS):
                s = s + ex[e]
            r = 1.0 / s
            for e in range(NUM_EXPERTS):
                plsc.store_scatter(obuf, [rows[g], e_idx[e]], ex[e] * r)

        pltpu.sync_copy(obuf, o_hbm.at[pl.ds(base + c * CHUNK, CHUNK)])


def kernel(x, W, b):
    W2 = jnp.concatenate([W, W], axis=1)
    b2 = jnp.tile(b.reshape(NUM_EXPERTS, 1), (1, L))
    mesh = plsc.VectorSubcoreMesh(core_axis_name="c", subcore_axis_name="s")
    f = pl.kernel(
        _sc_body,
        out_type=jax.ShapeDtypeStruct((N_TOKENS, NUM_EXPERTS), jnp.float32),
        mesh=mesh,
        scratch_types=[
            pltpu.VMEM((CHUNK, INPUT_DIM), jnp.float32),
            pltpu.VMEM((CHUNK, INPUT_DIM), jnp.float32),
            pltpu.VMEM((INPUT_DIM, 2 * NUM_EXPERTS), jnp.float32),
            pltpu.VMEM((NUM_EXPERTS, L), jnp.float32),
            pltpu.VMEM((CHUNK, NUM_EXPERTS), jnp.float32),
            pltpu.SemaphoreType.DMA,
            pltpu.SemaphoreType.DMA,
        ],
        compiler_params=pltpu.CompilerParams(
            needs_layout_passes=False, use_tc_tiling_on_sc=False),
    )
    return f(x, W2, b2)
